# speedup vs baseline: 3.3458x; 3.3458x over previous
import functools

import jax
import jax.numpy as jnp
from jax import lax
from jax.experimental import pallas as pl
from jax.experimental.pallas import tpu as pltpu
from jax.experimental.pallas import tpu_sc as plsc

VOCAB = 100000
HIDDEN = 128
NUM_IDS = 4096 * 50

_INFO = plsc.get_sparse_core_info()
NC = _INFO.num_cores
NS = _INFO.num_subcores
NW = NC * NS
B_PER_W = NUM_IDS // NW
CHUNK = 128
N_CHUNKS = B_PER_W // CHUNK
NBUF = 5


def _sc_body(ids_hbm, table_hbm, out_hbm, idx_v, rows_v, gsem, ssem):
    wid = lax.axis_index("s") * NC + lax.axis_index("c")
    pltpu.sync_copy(ids_hbm.at[wid], idx_v)

    def outer(i, carry):
        g = i * NBUF
        copies = []
        for b in range(NBUF):
            copies.append(
                pltpu.async_copy(table_hbm.at[idx_v.at[g + b]], rows_v.at[b], gsem)
            )
        for b in range(NBUF):
            copies[b].wait()
            pltpu.async_copy(rows_v.at[b], out_hbm.at[wid, g + b], ssem)
        for b in range(NBUF):
            pltpu.make_async_copy(rows_v.at[b], out_hbm.at[wid, g + b], ssem).wait()
        return carry

    lax.fori_loop(0, N_CHUNKS // NBUF, outer, 0)


@jax.jit
def _emb(ids, table):
    mesh = plsc.VectorSubcoreMesh(core_axis_name="c", subcore_axis_name="s")
    k = functools.partial(
        pl.kernel,
        mesh=mesh,
        out_type=jax.ShapeDtypeStruct((NW, N_CHUNKS, CHUNK, HIDDEN), jnp.float32),
        scratch_types=[
            pltpu.VMEM((N_CHUNKS, CHUNK), jnp.int32),
            pltpu.VMEM((NBUF, CHUNK, HIDDEN), jnp.float32),
            pltpu.SemaphoreType.DMA,
            pltpu.SemaphoreType.DMA,
        ],
    )(_sc_body)
    return k(ids, table)


def kernel(input_ids, table):
    ids = input_ids.reshape(NW, N_CHUNKS, CHUNK).astype(jnp.int32)
    out = _emb(ids, table)
    return out.reshape(*input_ids.shape, HIDDEN)
